# Initial kernel scaffold; baseline (speedup 1.0000x reference)
import jax
import jax.numpy as jnp
from jax.experimental import pallas as pl


def kernel(x, edge_index, W1, b1, W2, b2, Wm1, bm1, Wm2, bm2):
    raise NotImplementedError("write your pallas kernel here")


# speedup vs baseline: 6.7328x; 6.7328x over previous
import functools

import jax
import jax.numpy as jnp
from jax import lax
from jax.experimental import pallas as pl
from jax.experimental.pallas import tpu as pltpu
from jax.experimental.pallas import tpu_sc as plsc

F32 = jnp.float32
CHUNK = 128
DEGW = 16
LANE = 16


def _sizes(total, step):
    out = []
    while total > 0:
        out.append(min(step, total))
        total -= out[-1]
    return out


def _mesh():
    return plsc.VectorSubcoreMesh(core_axis_name="c", subcore_axis_name="s")


def _zero_rows(buf, rows, width):
    per_row = width // LANE

    def body(i, _):
        r = i // per_row
        c = (i % per_row) * LANE
        buf[r, pl.ds(c, LANE)] = jnp.zeros((LANE,), F32)
        return 0

    lax.fori_loop(0, rows * per_row, body, 0)


def _build_deg(nw, ns, nch, npad):
    rpt = npad // ns
    epw = nch * CHUNK

    @functools.partial(
        pl.kernel,
        out_type=jax.ShapeDtypeStruct((nw // ns * npad, DEGW), F32),
        mesh=_mesh(),
        scratch_types=[
            pltpu.VMEM((epw,), jnp.int32),
            pltpu.VMEM((CHUNK, DEGW), F32),
            pltpu.VMEM_SHARED((npad, DEGW), F32),
        ],
    )
    def deg_k(col_hbm, out_hbm, idx_v, ones_v, acc_sh):
        cid = lax.axis_index("c")
        sid = lax.axis_index("s")
        wid = cid * ns + sid
        base = sid * rpt

        _zero_rows(ones_v, CHUNK, DEGW)
        off = 0
        for sz in _sizes(rpt, CHUNK):
            pltpu.sync_copy(ones_v.at[pl.ds(0, sz)],
                            acc_sh.at[pl.ds(base + off, sz)])
            off += sz

        def fill_body(i, _):
            ones_v[i // (DEGW // LANE),
                   pl.ds((i % (DEGW // LANE)) * LANE, LANE)] = (
                jnp.full((LANE,), 1.0, F32))
            return 0
        lax.fori_loop(0, CHUNK * (DEGW // LANE), fill_body, 0)

        pltpu.sync_copy(col_hbm.at[pl.ds(wid * epw, epw)], idx_v)
        plsc.subcore_barrier()

        def chunk_body(j, _):
            pltpu.sync_copy(ones_v,
                            acc_sh.at[idx_v.at[pl.ds(j * CHUNK, CHUNK)]],
                            add=True)
            return 0
        lax.fori_loop(0, nch, chunk_body, 0)

        plsc.subcore_barrier()
        pltpu.sync_copy(acc_sh.at[pl.ds(base, rpt)],
                        out_hbm.at[pl.ds(cid * npad + base, rpt)])

    return deg_k


def _build_prop(nw, ns, nch, npad):
    w = 128
    sub = CHUNK // 2
    rpt = npad // ns

    @functools.partial(
        pl.kernel,
        out_type=jax.ShapeDtypeStruct((nw // ns, npad, w), F32),
        mesh=_mesh(),
        scratch_types=[
            pltpu.VMEM((nch, CHUNK), jnp.int32),
            pltpu.VMEM((nch, CHUNK), jnp.int32),
            pltpu.VMEM((2, sub, w), F32),
            pltpu.VMEM_SHARED((npad, w), F32),
            pltpu.SemaphoreType.DMA,
            pltpu.SemaphoreType.DMA,
        ],
    )
    def prop_k(table_hbm, row_hbm, col_hbm, out_hbm,
               ri_v, ci_v, gbuf, acc_sh, sem0, sem1):
        cid = lax.axis_index("c")
        sid = lax.axis_index("s")
        wid = cid * ns + sid
        base = sid * rpt
        sems = (sem0, sem1)

        _zero_rows(gbuf.at[0], sub, w)
        off = 0
        for sz in _sizes(rpt, sub):
            pltpu.sync_copy(gbuf.at[0, pl.ds(0, sz)],
                            acc_sh.at[pl.ds(base + off, sz)])
            off += sz

        pltpu.sync_copy(row_hbm.at[wid], ri_v)
        pltpu.sync_copy(col_hbm.at[wid], ci_v)
        plsc.subcore_barrier()

        pltpu.async_copy(table_hbm.at[ri_v.at[0, pl.ds(0, sub)]],
                         gbuf.at[0], sem0)
        pltpu.async_copy(table_hbm.at[ri_v.at[0, pl.ds(sub, sub)]],
                         gbuf.at[1], sem1)

        def chunk_body(p, _):
            for s in range(2):
                lo = s * sub
                pltpu.make_async_copy(
                    table_hbm.at[ri_v.at[p, pl.ds(lo, sub)]],
                    gbuf.at[s], sems[s]).wait()
                pltpu.sync_copy(gbuf.at[s],
                                acc_sh.at[ci_v.at[p, pl.ds(lo, sub)]],
                                add=True)

                @pl.when(p + 1 < nch)
                def _():
                    pltpu.async_copy(
                        table_hbm.at[ri_v.at[p + 1, pl.ds(lo, sub)]],
                        gbuf.at[s], sems[s])
            return 0
        lax.fori_loop(0, nch, chunk_body, 0)

        plsc.subcore_barrier()
        pltpu.sync_copy(acc_sh.at[pl.ds(base, rpt)],
                        out_hbm.at[cid, pl.ds(base, rpt)])

    return prop_k


def _build_mlp(nw, ns, nch, npad, h):
    w = 128
    ghl = h // LANE

    @functools.partial(
        pl.kernel,
        out_type=jax.ShapeDtypeStruct((nw, nch, CHUNK, LANE), F32),
        mesh=_mesh(),
        scratch_types=[
            pltpu.VMEM((nch, CHUNK), jnp.int32),
            pltpu.VMEM((nch, CHUNK), jnp.int32),
            pltpu.VMEM((2, CHUNK, w), F32),
            pltpu.VMEM((2, CHUNK, w), F32),
            pltpu.VMEM((ghl, LANE), F32),
            pltpu.VMEM((CHUNK, LANE), F32),
            pltpu.SemaphoreType.DMA,
            pltpu.SemaphoreType.DMA,
            pltpu.SemaphoreType.DMA,
            pltpu.SemaphoreType.DMA,
        ],
    )
    def mlp_k(t_hbm, row_hbm, col_hbm, w_hbm, out_hbm,
              ri_v, ci_v, bufa, bufb, w_v, p_v,
              sa0, sa1, sb0, sb1):
        cid = lax.axis_index("c")
        sid = lax.axis_index("s")
        wid = cid * ns + sid
        sas = (sa0, sa1)
        sbs = (sb0, sb1)

        pltpu.sync_copy(row_hbm.at[wid], ri_v)
        pltpu.sync_copy(col_hbm.at[wid], ci_v)
        pltpu.sync_copy(w_hbm, w_v)

        wg = [w_v[g] for g in range(ghl)]

        pltpu.async_copy(t_hbm.at[ri_v.at[0]], bufa.at[0], sa0)
        pltpu.async_copy(t_hbm.at[ci_v.at[0]], bufb.at[0], sb0)
        pltpu.async_copy(t_hbm.at[ri_v.at[1]], bufa.at[1], sa1)
        pltpu.async_copy(t_hbm.at[ci_v.at[1]], bufb.at[1], sb1)

        def pair_body(p, _):
            for s in range(2):
                j = 2 * p + s
                pltpu.make_async_copy(
                    t_hbm.at[ri_v.at[j]], bufa.at[s], sas[s]).wait()
                pltpu.make_async_copy(
                    t_hbm.at[ci_v.at[j]], bufb.at[s], sbs[s]).wait()

                def edge_body(e, _):
                    acc = jnp.zeros((LANE,), F32)
                    for g in range(ghl):
                        va = bufa[s, e, pl.ds(g * LANE, LANE)]
                        vb = bufb[s, e, pl.ds(h + g * LANE, LANE)]
                        acc = acc + jnp.maximum(va + vb, 0.0) * wg[g]
                    p_v[e] = acc
                    return 0
                lax.fori_loop(0, CHUNK, edge_body, 0)
                pltpu.sync_copy(p_v, out_hbm.at[wid, j])

                @pl.when(j + 2 < nch)
                def _():
                    pltpu.async_copy(
                        t_hbm.at[ri_v.at[j + 2]], bufa.at[s], sas[s])
                    pltpu.async_copy(
                        t_hbm.at[ci_v.at[j + 2]], bufb.at[s], sbs[s])
            return 0
        lax.fori_loop(0, nch // 2, pair_body, 0)

    return mlp_k


def _tc_stage_d(part, b2):
    ep = part.shape[0]
    rb = ep // 32

    def body(p_ref, b_ref, res_ref):
        res_ref[...] = 1.0 / (
            1.0 + jnp.exp(-(jnp.sum(p_ref[...], axis=1, keepdims=True)
                            + b_ref[0, 0])))

    return pl.pallas_call(
        body,
        grid=(ep // rb,),
        in_specs=[
            pl.BlockSpec((rb, LANE), lambda i: (i, 0)),
            pl.BlockSpec((1, 1), lambda i: (0, 0)),
        ],
        out_specs=pl.BlockSpec((rb, 1), lambda i: (i, 0)),
        out_shape=jax.ShapeDtypeStruct((ep, 1), F32),
    )(part, b2)


def _tc_stage_a(x, w1, d0, d1):
    n, f = x.shape
    h = w1.shape[1]
    rb = 2000

    def body(x_ref, w_ref, d0_ref, d1_ref, g0_ref, dis_ref, dis2_ref):
        deg = d0_ref[:, 0:1] + d1_ref[:, 0:1]
        pos = deg > 0.0
        dis = jnp.where(pos, lax.rsqrt(jnp.where(pos, deg, 1.0)), 0.0)
        dis2 = lax.rsqrt(deg + 1.0)
        hh = jnp.dot(x_ref[...], w_ref[...], preferred_element_type=F32)
        g0_ref[...] = dis * hh
        dis_ref[...] = dis
        dis2_ref[...] = dis2

    return pl.pallas_call(
        body,
        grid=(n // rb,),
        in_specs=[
            pl.BlockSpec((rb, f), lambda i: (i, 0)),
            pl.BlockSpec((f, h), lambda i: (0, 0)),
            pl.BlockSpec((rb, DEGW), lambda i: (i, 0)),
            pl.BlockSpec((rb, DEGW), lambda i: (i, 0)),
        ],
        out_specs=[
            pl.BlockSpec((rb, h), lambda i: (i, 0)),
            pl.BlockSpec((rb, 1), lambda i: (i, 0)),
            pl.BlockSpec((rb, 1), lambda i: (i, 0)),
        ],
        out_shape=[
            jax.ShapeDtypeStruct((n, h), F32),
            jax.ShapeDtypeStruct((n, 1), F32),
            jax.ShapeDtypeStruct((n, 1), F32),
        ],
    )(x, w1, d0, d1)


def _tc_stage_b(s1a, s1b, dis, dis2, b1, w2):
    n, h = s1a.shape
    rb = 2000

    def body(sa_ref, sb_ref, dis_ref, dis2_ref, b_ref, w_ref, g1_ref):
        h1 = jnp.maximum(dis_ref[...] * (sa_ref[...] + sb_ref[...])
                         + b_ref[...], 0.0)
        g1_ref[...] = dis2_ref[...] * jnp.dot(
            h1, w_ref[...], preferred_element_type=F32)

    return pl.pallas_call(
        body,
        grid=(n // rb,),
        in_specs=[
            pl.BlockSpec((rb, h), lambda i: (i, 0)),
            pl.BlockSpec((rb, h), lambda i: (i, 0)),
            pl.BlockSpec((rb, 1), lambda i: (i, 0)),
            pl.BlockSpec((rb, 1), lambda i: (i, 0)),
            pl.BlockSpec((1, h), lambda i: (0, 0)),
            pl.BlockSpec((h, h), lambda i: (0, 0)),
        ],
        out_specs=pl.BlockSpec((rb, h), lambda i: (i, 0)),
        out_shape=jax.ShapeDtypeStruct((n, h), F32),
    )(s1a, s1b, dis, dis2, b1, w2)


def _tc_stage_c(s2a, s2b, g1, dis2, b2, wa, wb, bm1):
    n, h = g1.shape
    rb = 2000

    def body(sa_ref, sb_ref, g1_ref, dis2_ref, b_ref, wa_ref, wb_ref,
             bm_ref, ha_ref, hb_ref):
        h2 = jnp.maximum(
            dis2_ref[...] * (sa_ref[...] + sb_ref[...] + g1_ref[...])
            + b_ref[...], 0.0)
        ha_ref[...] = jnp.dot(h2, wa_ref[...],
                              preferred_element_type=F32) + bm_ref[...]
        hb_ref[...] = jnp.dot(h2, wb_ref[...], preferred_element_type=F32)

    return pl.pallas_call(
        body,
        grid=(n // rb,),
        in_specs=[
            pl.BlockSpec((rb, h), lambda i: (i, 0)),
            pl.BlockSpec((rb, h), lambda i: (i, 0)),
            pl.BlockSpec((rb, h), lambda i: (i, 0)),
            pl.BlockSpec((rb, 1), lambda i: (i, 0)),
            pl.BlockSpec((1, h), lambda i: (0, 0)),
            pl.BlockSpec((h, h), lambda i: (0, 0)),
            pl.BlockSpec((h, h), lambda i: (0, 0)),
            pl.BlockSpec((1, h), lambda i: (0, 0)),
        ],
        out_specs=[
            pl.BlockSpec((rb, h), lambda i: (i, 0)),
            pl.BlockSpec((rb, h), lambda i: (i, 0)),
        ],
        out_shape=[
            jax.ShapeDtypeStruct((n, h), F32),
            jax.ShapeDtypeStruct((n, h), F32),
        ],
    )(s2a, s2b, g1, dis2, b2, wa, wb, bm1)


def kernel(x, edge_index, W1, b1, W2, b2, Wm1, bm1, Wm2, bm2):
    n, f = x.shape
    h = W1.shape[1]
    e = edge_index.shape[1]

    info = plsc.get_sparse_core_info()
    nc, ns = info.num_cores, info.num_subcores
    nw = nc * ns

    per = nw * CHUNK
    nch = -(-e // per)
    if nch % 2:
        nch += 1
    ep = nch * per
    pad_idx = jnp.full((ep - e,), n, jnp.int32)
    rowp = jnp.concatenate([edge_index[0], pad_idx]).reshape(nw, nch, CHUNK)
    colp = jnp.concatenate([edge_index[1], pad_idx]).reshape(nw, nch, CHUNK)

    npad = -(-(n + 1) // (ns * 8)) * (ns * 8)

    deg_parts = _build_deg(nw, ns, nch, npad)(
        colp.reshape(-1)).reshape(nc, npad, DEGW)
    d0 = deg_parts[0, :n]
    d1 = deg_parts[1, :n]

    g0, dis, dis2 = _tc_stage_a(x, W1, d0, d1)

    prop = _build_prop(nw, ns, nch, npad)
    pad_tbl = ((0, npad - n), (0, 128 - h))
    s1 = prop(jnp.pad(g0, pad_tbl), rowp, colp)
    g1 = _tc_stage_b(s1[0, :n, :h], s1[1, :n, :h], dis, dis2,
                     b1.reshape(1, h), W2)

    s2 = prop(jnp.pad(g1, pad_tbl), rowp, colp)
    ha, hb = _tc_stage_c(s2[0, :n, :h], s2[1, :n, :h], g1, dis2,
                         b2.reshape(1, h), Wm1[:h], Wm1[h:],
                         bm1.reshape(1, h))

    t_mlp = jnp.pad(jnp.concatenate([ha, hb], axis=1),
                    ((0, npad - n), (0, 0)))
    part = _build_mlp(nw, ns, nch, npad, h)(
        t_mlp, rowp, colp, Wm2.reshape(h // LANE, LANE))
    res = _tc_stage_d(part.reshape(ep, LANE), bm2.reshape(1, 1))
    return res[:e]
